# baseline (speedup 1.0000x reference)
import jax
import jax.numpy as jnp
from jax.experimental import pallas as pl
from jax.experimental.pallas import tpu as pltpu

_T, _D, _DO, _E = 4096, 1024, 1024, 8
_BT = 256


def _moe_body(x_ref, wr_ref, br_ref, we_ref, be_ref, o_ref, web_ref):
    @pl.when(pl.program_id(0) == 0)
    def _cast_weights():
        for e in range(_E):
            web_ref[pl.ds(e * _D, _D), :] = we_ref[e].astype(jnp.bfloat16)

    x = x_ref[...]
    logits = jnp.dot(x, wr_ref[...], preferred_element_type=jnp.float32)
    logits = logits + br_ref[...]
    m = jnp.max(logits, axis=-1, keepdims=True)
    p = jnp.exp(logits - m)
    w = p / jnp.sum(p, axis=-1, keepdims=True)
    rank = jnp.zeros_like(w)
    for j in range(_E):
        rank = rank + (w[:, j : j + 1] > w).astype(jnp.float32)
    sw = jnp.where(rank < 2.0, w, 0.0)
    acc = jnp.dot(sw, be_ref[...], preferred_element_type=jnp.float32)
    xw = jnp.concatenate(
        [(x * sw[:, e : e + 1]).astype(jnp.bfloat16) for e in range(_E)], axis=1
    )
    acc = acc + jnp.dot(xw, web_ref[...], preferred_element_type=jnp.float32)
    o_ref[...] = acc


def kernel(x, Wr, br, We, be):
    br2 = br.reshape(1, _E)
    return pl.pallas_call(
        _moe_body,
        grid=(_T // _BT,),
        in_specs=[
            pl.BlockSpec((_BT, _D), lambda i: (i, 0)),
            pl.BlockSpec((_D, _E), lambda i: (0, 0)),
            pl.BlockSpec((1, _E), lambda i: (0, 0)),
            pl.BlockSpec((_E, _D, _DO), lambda i: (0, 0, 0)),
            pl.BlockSpec((_E, _DO), lambda i: (0, 0)),
        ],
        out_specs=pl.BlockSpec((_BT, _DO), lambda i: (i, 0)),
        out_shape=jax.ShapeDtypeStruct((_T, _DO), jnp.float32),
        scratch_shapes=[pltpu.VMEM((_E * _D, _DO), jnp.bfloat16)],
        compiler_params=pltpu.CompilerParams(
            dimension_semantics=("arbitrary",),
        ),
    )(x, Wr, br2, We, be)


# speedup vs baseline: 1.0654x; 1.0654x over previous
import jax
import jax.numpy as jnp
from jax.experimental import pallas as pl
from jax.experimental.pallas import tpu as pltpu

_T, _D, _DO, _E = 4096, 1024, 1024, 8
_BT = 256
_NT = _T // _BT


def _router(x, wr, br):
    logits = jnp.dot(x, wr, preferred_element_type=jnp.float32) + br
    m = jnp.max(logits, axis=-1, keepdims=True)
    p = jnp.exp(logits - m)
    w = p / jnp.sum(p, axis=-1, keepdims=True)
    rank = jnp.zeros_like(w)
    for j in range(_E):
        rank = rank + (w[:, j : j + 1] > w).astype(jnp.float32)
    return jnp.where(rank < 2.0, w, 0.0)


def _moe_body(xn_ref, x_ref, wr_ref, br_ref, we_ref, be_ref, o_ref,
              web_ref, sw_ref):
    i = pl.program_id(0)

    @pl.when(i == 0)
    def _prologue():
        for e in range(_E):
            web_ref[pl.ds(e * _D, _D), :] = we_ref[e].astype(jnp.bfloat16)
        sw_ref[0] = _router(x_ref[...], wr_ref[...], br_ref[...])

    x = x_ref[...]
    sw = sw_ref[i % 2]
    acc = jnp.dot(sw, be_ref[...], preferred_element_type=jnp.float32)
    xb = x.astype(jnp.bfloat16)
    for e in range(_E):
        y = jnp.dot(xb, web_ref[pl.ds(e * _D, _D), :],
                    preferred_element_type=jnp.float32)
        acc = acc + sw[:, e : e + 1] * y
    o_ref[...] = acc
    sw_ref[(i + 1) % 2] = _router(xn_ref[...], wr_ref[...], br_ref[...])


def kernel(x, Wr, br, We, be):
    br2 = br.reshape(1, _E)
    return pl.pallas_call(
        _moe_body,
        grid=(_NT,),
        in_specs=[
            pl.BlockSpec((_BT, _D), lambda i: (jnp.minimum(i + 1, _NT - 1), 0)),
            pl.BlockSpec((_BT, _D), lambda i: (i, 0)),
            pl.BlockSpec((_D, _E), lambda i: (0, 0)),
            pl.BlockSpec((1, _E), lambda i: (0, 0)),
            pl.BlockSpec((_E, _D, _DO), lambda i: (0, 0, 0)),
            pl.BlockSpec((_E, _DO), lambda i: (0, 0)),
        ],
        out_specs=pl.BlockSpec((_BT, _DO), lambda i: (i, 0)),
        out_shape=jax.ShapeDtypeStruct((_T, _DO), jnp.float32),
        scratch_shapes=[
            pltpu.VMEM((_E * _D, _DO), jnp.bfloat16),
            pltpu.VMEM((2, _BT, _E), jnp.float32),
        ],
        compiler_params=pltpu.CompilerParams(
            dimension_semantics=("arbitrary",),
        ),
    )(x, x, Wr, br2, We, be)
